# baseline (speedup 1.0000x reference)
import jax
import jax.numpy as jnp
from jax import lax
from jax.experimental import pallas as pl
from jax.experimental.pallas import tpu as pltpu
from jax.experimental.pallas import tpu_sc as plsc

NC = 2
NS = 16
L = 16
NW = NC * NS

B = 16384
D = 32
N_EX = 100000
N_EX_PAD = 782 * 128
BPW = B // NW
CH = 128
NCHUNK = BPW // CH
GPC = CH // L
NBUF = 2


def _mirt_body(user_hbm, item_hbm, theta_hbm, a_hbm, b_hbm, out_hbm, *scr):
    idx_u = scr[0:NCHUNK]
    idx_i = scr[NCHUNK:2 * NCHUNK]
    idx_ut = scr[2 * NCHUNK:3 * NCHUNK]
    idx_it = scr[3 * NCHUNK:4 * NCHUNK]
    idx_ib = scr[4 * NCHUNK:5 * NCHUNK]
    o = 5 * NCHUNK
    th_v = scr[o:o + NBUF]
    a_v = scr[o + NBUF:o + 2 * NBUF]
    b_v = scr[o + 2 * NBUF:o + 3 * NBUF]
    out_v = scr[o + 3 * NBUF]
    o = o + 3 * NBUF + 1
    sem_t = scr[o:o + NBUF]
    sem_a = scr[o + NBUF:o + 2 * NBUF]
    sem_b = scr[o + 2 * NBUF:o + 3 * NBUF]

    wid = lax.axis_index("s") * NC + lax.axis_index("c")
    base = wid * BPW

    for c in range(NCHUNK):
        pltpu.sync_copy(user_hbm.at[pl.ds(base + c * CH, CH)], idx_u[c])
        pltpu.sync_copy(item_hbm.at[pl.ds(base + c * CH, CH)], idx_i[c])
    for c in range(NCHUNK):
        for k in range(GPC):
            s = pl.ds(k * L, L)
            u = idx_u[c][s]
            it = idx_i[c][s]
            idx_ut[c][s] = u >> 2
            idx_it[c][s] = it >> 2
            idx_ib[c][s] = it >> 7

    def fire(c):
        buf = c % NBUF
        ct = pltpu.async_copy(theta_hbm.at[idx_ut[c]], th_v[buf], sem_t[buf])
        ca = pltpu.async_copy(a_hbm.at[idx_it[c]], a_v[buf], sem_a[buf])
        cb = pltpu.async_copy(b_hbm.at[idx_ib[c]], b_v[buf], sem_b[buf])
        return ct, ca, cb

    copies = [fire(0), fire(1)]

    lane = lax.iota(jnp.int32, L)

    for c in range(NCHUNK):
        buf = c % NBUF
        ct, ca, cb = copies[c]
        ct.wait()
        ca.wait()
        cb.wait()

        def group_body(g, _, c=c, buf=buf):
            rows = g * L + lane
            s = pl.ds(g * L, L)
            u = idx_u[c][s]
            it = idx_i[c][s]
            ucol = (u & 3) << 5
            icol = (it & 3) << 5
            acc = jnp.zeros((L,), jnp.float32)
            for j in range(D):
                t = plsc.load_gather(th_v[buf], [rows, ucol + j])
                av = plsc.load_gather(a_v[buf], [rows, icol + j])
                acc = acc + (t + t) / (1.0 + jnp.exp(-av))
            bv = plsc.load_gather(b_v[buf], [rows, it & 127])
            res = 1.0 / (1.0 + jnp.exp(bv - acc))
            out_v[pl.ds(c * CH + g * L, L)] = res
            return 0

        lax.fori_loop(0, GPC, group_body, 0)
        if c + NBUF < NCHUNK:
            copies.append(fire(c + NBUF))

    pltpu.sync_copy(out_v, out_hbm.at[pl.ds(base, BPW)])


@jax.jit
def kernel(user, item, theta_table, a_table, b_table):
    theta_blocks = theta_table.reshape(-1, 4 * D)
    a_blocks = a_table.reshape(-1, 4 * D)
    b_flat = b_table.reshape(-1)
    b_blocks = jnp.pad(b_flat, (0, N_EX_PAD - N_EX)).reshape(-1, 128)
    mesh = plsc.VectorSubcoreMesh(
        core_axis_name="c", subcore_axis_name="s",
        num_cores=NC, num_subcores=NS)
    scratch = (
        [pltpu.VMEM((CH,), jnp.int32) for _ in range(5 * NCHUNK)]
        + [pltpu.VMEM((CH, 4 * D), jnp.float32) for _ in range(NBUF)]
        + [pltpu.VMEM((CH, 4 * D), jnp.float32) for _ in range(NBUF)]
        + [pltpu.VMEM((CH, 128), jnp.float32) for _ in range(NBUF)]
        + [pltpu.VMEM((BPW,), jnp.float32)]
        + [pltpu.SemaphoreType.DMA for _ in range(3 * NBUF)]
    )
    f = pl.kernel(
        _mirt_body,
        out_type=jax.ShapeDtypeStruct((B,), jnp.float32),
        mesh=mesh,
        compiler_params=pltpu.CompilerParams(needs_layout_passes=False),
        scratch_types=scratch,
    )
    return f(user, item, theta_blocks, a_blocks, b_blocks)


# speedup vs baseline: 1.4696x; 1.4696x over previous
import jax
import jax.numpy as jnp
from jax import lax
from jax.experimental import pallas as pl
from jax.experimental.pallas import tpu as pltpu
from jax.experimental.pallas import tpu_sc as plsc

NC = 2
NS = 16
L = 16
NW = NC * NS

B = 16384
D = 32
N_EX = 100000
N_EX_PAD = 782 * 128
BPW = B // NW
NG = BPW // L
SLOT_G = 4
NSLOT = SLOT_G * L
BCH = 128
NBCH = BPW // BCH


def _mirt_body(user_hbm, item_hbm, theta_hbm, a_hbm, b_hbm, out_hbm, *scr):
    u_all, it_all = scr[0], scr[1]
    idx_ib = scr[2]
    th_ring = scr[3]
    a_ring = scr[4]
    b_v = scr[5]
    out_v = scr[6]
    sem_t = scr[7:7 + SLOT_G]
    sem_a = scr[7 + SLOT_G:7 + 2 * SLOT_G]
    sem_b = scr[7 + 2 * SLOT_G]

    wid = lax.axis_index("s") * NC + lax.axis_index("c")
    base = wid * BPW

    pltpu.sync_copy(user_hbm.at[pl.ds(base, BPW)], u_all)
    pltpu.sync_copy(item_hbm.at[pl.ds(base, BPW)], it_all)

    for k in range(BPW // L):
        c, hi = divmod(k, BCH // L)
        idx_ib[c, pl.ds(hi * L, L)] = it_all[pl.ds(k * L, L)] >> 7
    b_copies = [
        pltpu.async_copy(b_hbm.at[idx_ib.at[c]],
                         b_v.at[pl.ds(c * BCH, BCH), :], sem_b)
        for c in range(NBCH)
    ]

    prime = []
    for g in range(SLOT_G):
        uvec = u_all[pl.ds(g * L, L)]
        itvec = it_all[pl.ds(g * L, L)]
        for i in range(L):
            row = g * L + i
            ct = pltpu.async_copy(theta_hbm.at[uvec[i]], th_ring.at[row],
                                  sem_t[g])
            ca = pltpu.async_copy(a_hbm.at[itvec[i]], a_ring.at[row],
                                  sem_a[g])
            prime.append((ct, ca))

    for bc in b_copies:
        bc.wait()

    lane = lax.iota(jnp.int32, L)

    def super_body(p, _):
        for gi in range(SLOT_G):
            g = p * SLOT_G + gi
            for i in range(L):
                prime[gi * L + i][0].wait()
                prime[gi * L + i][1].wait()

            slots = gi * L + lane
            s = pl.ds(g * L, L)
            it = it_all[s]
            acc = jnp.zeros((L,), jnp.float32)
            for j in range(D):
                jj = jnp.full((L,), j, jnp.int32)
                t = plsc.load_gather(th_ring, [slots, jj])
                av = plsc.load_gather(a_ring, [slots, jj])
                acc = acc + (t + t) / (1.0 + jnp.exp(-av))
            bv = plsc.load_gather(b_v, [g * L + lane, it & 127])
            res = 1.0 / (1.0 + jnp.exp(bv - acc))
            out_v[pl.ds(g * L, L)] = res

            @pl.when(g + SLOT_G < NG)
            def _(g=g, gi=gi):
                gn = g + SLOT_G
                uvec = u_all[pl.ds(gn * L, L)]
                itvec = it_all[pl.ds(gn * L, L)]
                for i in range(L):
                    slot = gi * L + i
                    pltpu.async_copy(theta_hbm.at[uvec[i]],
                                     th_ring.at[slot], sem_t[gi])
                    pltpu.async_copy(a_hbm.at[itvec[i]],
                                     a_ring.at[slot], sem_a[gi])

        return 0

    lax.fori_loop(0, NG // SLOT_G, super_body, 0)

    pltpu.sync_copy(out_v, out_hbm.at[pl.ds(base, BPW)])


@jax.jit
def kernel(user, item, theta_table, a_table, b_table):
    b_flat = b_table.reshape(-1)
    b_blocks = jnp.pad(b_flat, (0, N_EX_PAD - N_EX)).reshape(-1, 128)
    mesh = plsc.VectorSubcoreMesh(
        core_axis_name="c", subcore_axis_name="s",
        num_cores=NC, num_subcores=NS)
    scratch = (
        [pltpu.VMEM((BPW,), jnp.int32) for _ in range(2)]
        + [pltpu.VMEM((NBCH, BCH), jnp.int32)]
        + [pltpu.VMEM((NSLOT, D), jnp.float32) for _ in range(2)]
        + [pltpu.VMEM((BPW, 128), jnp.float32)]
        + [pltpu.VMEM((BPW,), jnp.float32)]
        + [pltpu.SemaphoreType.DMA for _ in range(2 * SLOT_G + 1)]
    )
    f = pl.kernel(
        _mirt_body,
        out_type=jax.ShapeDtypeStruct((B,), jnp.float32),
        mesh=mesh,
        compiler_params=pltpu.CompilerParams(needs_layout_passes=False),
        scratch_types=scratch,
    )
    return f(user, item, theta_table, a_table, b_blocks)
